# baseline (speedup 1.0000x reference)
import functools

import jax
import jax.numpy as jnp
from jax import lax
from jax.experimental import pallas as pl
from jax.experimental.pallas import tpu as pltpu
from jax.experimental.pallas import tpu_sc as plsc

B = 16384
D = 64
K = 5
KP = 8
NC = 2
NS = 16
NW = NC * NS
BPW = B // NW
CH = 64
NCH = BPW // CH
GL = 8

_mesh = plsc.VectorSubcoreMesh(core_axis_name="c", subcore_axis_name="s",
                               num_cores=NC)


def _gather_body(is3d):
    def body(g3, s3, tab, out_hbm, g_v, s_v, grp_v, rows_v, gsem, wsem):
        wid = lax.axis_index("s") * NC + lax.axis_index("c")
        base = wid * BPW

        pltpu.sync_copy(g3.at[wid], g_v)
        pltpu.sync_copy(s3.at[wid], s_v)

        for c in range(NCH):
            def fire_body(g, carry, c=c):
                gv = g_v[c, pl.ds(g * 16, 16)]
                for lane in range(16):
                    if is3d:
                        src = tab.at[gv[lane]]
                    else:
                        start = pl.multiple_of(gv[lane] * GL, GL)
                        src = tab.at[pl.ds(start, GL)]
                    pltpu.async_copy(src, grp_v.at[g * 16 + lane], gsem)
                return carry

            lax.fori_loop(0, CH // 16, fire_body, 0)

            for j in range(CH):
                if is3d:
                    dummy = tab.at[0]
                else:
                    dummy = tab.at[pl.ds(0, GL)]
                pltpu.make_async_copy(dummy, grp_v.at[0], gsem).wait()

            def extract_body(g, carry, c=c):
                sv = s_v[c, pl.ds(g * 16, 16)]
                for lane in range(16):
                    j = g * 16 + lane
                    sub = sv[lane]
                    for q in range(D // 16):
                        sl = pl.ds(q * 16, 16)
                        rows_v[j, sl] = grp_v[j, sub, sl]
                return carry

            lax.fori_loop(0, CH // 16, extract_body, 0)
            pltpu.async_copy(
                rows_v, out_hbm.at[pl.ds(base + c * CH, CH)], wsem).wait()

    return body


_gather_scratch = [
    pltpu.VMEM((NCH, CH), jnp.int32),
    pltpu.VMEM((NCH, CH), jnp.int32),
    pltpu.VMEM((CH, GL, D), jnp.float32),
    pltpu.VMEM((CH, D), jnp.float32),
    pltpu.SemaphoreType.DMA,
    pltpu.SemaphoreType.DMA,
]

_sc_gather_rows = pl.kernel(
    _gather_body(True),
    mesh=_mesh,
    compiler_params=pltpu.CompilerParams(use_tc_tiling_on_sc=True),
    out_type=jax.ShapeDtypeStruct((B, D), jnp.float32),
    scratch_types=_gather_scratch,
)

_sc_gather_rows2d = pl.kernel(
    _gather_body(False),
    mesh=_mesh,
    compiler_params=pltpu.CompilerParams(use_tc_tiling_on_sc=True),
    out_type=jax.ShapeDtypeStruct((B, D), jnp.float32),
    scratch_types=_gather_scratch,
)


def _tc_body(u_ref, p_ref, w_ref, b_ref, o_ref):
    inter = u_ref[...] * p_ref[...]
    o_ref[...] = (
        jnp.dot(inter, w_ref[...], preferred_element_type=jnp.float32)
        + b_ref[...]
    )


_ROWS_BLK = 2048

_tc_logits = pl.pallas_call(
    _tc_body,
    grid=(B // _ROWS_BLK,),
    in_specs=[
        pl.BlockSpec((_ROWS_BLK, D), lambda i: (i, 0)),
        pl.BlockSpec((_ROWS_BLK, D), lambda i: (i, 0)),
        pl.BlockSpec((D, KP), lambda i: (0, 0)),
        pl.BlockSpec((1, KP), lambda i: (0, 0)),
    ],
    out_specs=pl.BlockSpec((_ROWS_BLK, KP), lambda i: (i, 0)),
    out_shape=jax.ShapeDtypeStruct((B, KP), jnp.float32),
)


def kernel(user, product, user_factors, product_factors, W, b):
    user = user.astype(jnp.int32)
    product = product.astype(jnp.int32)
    ug3 = (user >> 3).reshape(NW, NCH, CH)
    us3 = (user & 7).reshape(NW, NCH, CH)
    pg3 = (product >> 3).reshape(NW, NCH, CH)
    ps3 = (product & 7).reshape(NW, NCH, CH)

    uf8 = user_factors.reshape(125000, GL, D)
    u_rows = _sc_gather_rows(ug3, us3, uf8)
    p_rows = _sc_gather_rows2d(pg3, ps3, product_factors)

    wt = jnp.zeros((D, KP), jnp.float32).at[:, :K].set(W.T)
    bp = jnp.zeros((1, KP), jnp.float32).at[0, :K].set(b)
    out = _tc_logits(u_rows, p_rows, wt, bp)
    return out[:, :K]


# speedup vs baseline: 1.0837x; 1.0837x over previous
import functools

import jax
import jax.numpy as jnp
from jax import lax
from jax.experimental import pallas as pl
from jax.experimental.pallas import tpu as pltpu
from jax.experimental.pallas import tpu_sc as plsc

B = 16384
D = 64
K = 5
KP = 8
NC = 2
NS = 16
NW = NC * NS
BPW = B // NW
CH = 32
NCH = BPW // CH
GL = 8

_mesh = plsc.VectorSubcoreMesh(core_axis_name="c", subcore_axis_name="s",
                               num_cores=NC)


def _gather_body(is3d):
    def body(g3, s3, tab, out_hbm, g_v, s_v, grp_v0, grp_v1, rows_v,
             gsem0, gsem1, wsem):
        wid = lax.axis_index("s") * NC + lax.axis_index("c")
        base = wid * BPW
        bufs = [grp_v0, grp_v1]
        sems = [gsem0, gsem1]

        pltpu.sync_copy(g3.at[wid], g_v)
        pltpu.sync_copy(s3.at[wid], s_v)

        def fire(c, buf, sem):
            def fire_body(g, carry):
                gv = g_v[c, pl.ds(g * 16, 16)]
                for lane in range(16):
                    if is3d:
                        src = tab.at[gv[lane]]
                    else:
                        start = pl.multiple_of(gv[lane] * GL, GL)
                        src = tab.at[pl.ds(start, GL)]
                    pltpu.async_copy(src, buf.at[g * 16 + lane], sem)
                return carry

            lax.fori_loop(0, CH // 16, fire_body, 0)

        fire(0, bufs[0], sems[0])
        for c in range(NCH):
            if c + 1 < NCH:
                fire(c + 1, bufs[(c + 1) % 2], sems[(c + 1) % 2])
            buf = bufs[c % 2]
            for j in range(CH):
                if is3d:
                    dummy = tab.at[0]
                else:
                    dummy = tab.at[pl.ds(0, GL)]
                pltpu.make_async_copy(dummy, buf.at[0], sems[c % 2]).wait()

            def extract_body(g, carry, c=c, buf=buf):
                sv = s_v[c, pl.ds(g * 16, 16)]
                for lane in range(16):
                    j = g * 16 + lane
                    sub = sv[lane]
                    for q in range(D // 16):
                        sl = pl.ds(q * 16, 16)
                        rows_v[j, sl] = buf[j, sub, sl]
                return carry

            lax.fori_loop(0, CH // 16, extract_body, 0)
            pltpu.async_copy(
                rows_v, out_hbm.at[pl.ds(base + c * CH, CH)], wsem).wait()

    return body


_gather_scratch = [
    pltpu.VMEM((NCH, CH), jnp.int32),
    pltpu.VMEM((NCH, CH), jnp.int32),
    pltpu.VMEM((CH, GL, D), jnp.float32),
    pltpu.VMEM((CH, GL, D), jnp.float32),
    pltpu.VMEM((CH, D), jnp.float32),
    pltpu.SemaphoreType.DMA,
    pltpu.SemaphoreType.DMA,
    pltpu.SemaphoreType.DMA,
]

_sc_gather_rows = pl.kernel(
    _gather_body(True),
    mesh=_mesh,
    compiler_params=pltpu.CompilerParams(use_tc_tiling_on_sc=True),
    out_type=jax.ShapeDtypeStruct((B, D), jnp.float32),
    scratch_types=_gather_scratch,
)

_sc_gather_rows2d = pl.kernel(
    _gather_body(False),
    mesh=_mesh,
    compiler_params=pltpu.CompilerParams(use_tc_tiling_on_sc=True),
    out_type=jax.ShapeDtypeStruct((B, D), jnp.float32),
    scratch_types=_gather_scratch,
)


def _tc_body(u_ref, p_ref, w_ref, b_ref, o_ref):
    inter = u_ref[...] * p_ref[...]
    o_ref[...] = (
        jnp.dot(inter, w_ref[...], preferred_element_type=jnp.float32)
        + b_ref[...]
    )


_ROWS_BLK = 2048

_tc_logits = pl.pallas_call(
    _tc_body,
    grid=(B // _ROWS_BLK,),
    in_specs=[
        pl.BlockSpec((_ROWS_BLK, D), lambda i: (i, 0)),
        pl.BlockSpec((_ROWS_BLK, D), lambda i: (i, 0)),
        pl.BlockSpec((D, KP), lambda i: (0, 0)),
        pl.BlockSpec((1, KP), lambda i: (0, 0)),
    ],
    out_specs=pl.BlockSpec((_ROWS_BLK, KP), lambda i: (i, 0)),
    out_shape=jax.ShapeDtypeStruct((B, KP), jnp.float32),
)


def kernel(user, product, user_factors, product_factors, W, b):
    user = user.astype(jnp.int32)
    product = product.astype(jnp.int32)
    ug3 = (user >> 3).reshape(NW, NCH, CH)
    us3 = (user & 7).reshape(NW, NCH, CH)
    pg3 = (product >> 3).reshape(NW, NCH, CH)
    ps3 = (product & 7).reshape(NW, NCH, CH)

    uf8 = user_factors.reshape(125000, GL, D)
    pf8 = product_factors.reshape(125000, GL, D)
    u_rows = _sc_gather_rows(ug3, us3, uf8)
    p_rows = _sc_gather_rows(pg3, ps3, pf8)

    wt = jnp.zeros((D, KP), jnp.float32).at[:, :K].set(W.T)
    bp = jnp.zeros((1, KP), jnp.float32).at[0, :K].set(b)
    out = _tc_logits(u_rows, p_rows, wt, bp)
    return out[:, :K]


# speedup vs baseline: 1.0848x; 1.0009x over previous
import functools

import jax
import jax.numpy as jnp
from jax import lax
from jax.experimental import pallas as pl
from jax.experimental.pallas import tpu as pltpu
from jax.experimental.pallas import tpu_sc as plsc

B = 16384
D = 64
K = 5
KP = 8
NC = 2
NS = 16
NW = NC * NS
BPW = B // NW
CH = 32
NCH = BPW // CH
GL = 8

_mesh = plsc.VectorSubcoreMesh(core_axis_name="c", subcore_axis_name="s",
                               num_cores=NC)


def _gather_body(is3d):
    def body(g3, s3, tab, out_hbm, g_v, s_v, grp_v0, grp_v1, rows_v,
             gsem0, gsem1, wsem):
        wid = lax.axis_index("s") * NC + lax.axis_index("c")
        base = wid * BPW
        bufs = [grp_v0, grp_v1]
        sems = [gsem0, gsem1]

        pltpu.sync_copy(g3.at[wid], g_v)
        pltpu.sync_copy(s3.at[wid], s_v)

        def fire(c, buf, sem):
            def fire_body(g, carry):
                gv = g_v[c, pl.ds(g * 16, 16)]
                for lane in range(16):
                    if is3d:
                        src = tab.at[gv[lane]]
                    else:
                        start = pl.multiple_of(gv[lane] * GL, GL)
                        src = tab.at[pl.ds(start, GL)]
                    pltpu.async_copy(src, buf.at[g * 16 + lane], sem)
                return carry

            lax.fori_loop(0, CH // 16, fire_body, 0)

        fire(0, bufs[0], sems[0])
        for c in range(NCH):
            if c + 1 < NCH:
                fire(c + 1, bufs[(c + 1) % 2], sems[(c + 1) % 2])
            buf = bufs[c % 2]
            for j in range(CH):
                if is3d:
                    dummy = tab.at[0]
                else:
                    dummy = tab.at[pl.ds(0, GL)]
                pltpu.make_async_copy(dummy, buf.at[0], sems[c % 2]).wait()

            def extract_body(g, carry, c=c, buf=buf):
                sv = s_v[c, pl.ds(g * 16, 16)]
                for lane in range(16):
                    j = g * 16 + lane
                    sub = sv[lane]
                    for q in range(D // 16):
                        sl = pl.ds(q * 16, 16)
                        rows_v[j, sl] = buf[j, sub, sl]
                return carry

            lax.fori_loop(0, CH // 16, extract_body, 0)
            pltpu.async_copy(
                rows_v, out_hbm.at[pl.ds(base + c * CH, CH)], wsem).wait()

    return body


_gather_scratch = [
    pltpu.VMEM((NCH, CH), jnp.int32),
    pltpu.VMEM((NCH, CH), jnp.int32),
    pltpu.VMEM((CH, GL, D), jnp.float32),
    pltpu.VMEM((CH, GL, D), jnp.float32),
    pltpu.VMEM((CH, D), jnp.float32),
    pltpu.SemaphoreType.DMA,
    pltpu.SemaphoreType.DMA,
    pltpu.SemaphoreType.DMA,
]

_sc_gather_rows = pl.kernel(
    _gather_body(True),
    mesh=_mesh,
    compiler_params=pltpu.CompilerParams(use_tc_tiling_on_sc=True),
    out_type=jax.ShapeDtypeStruct((B, D), jnp.float32),
    scratch_types=_gather_scratch,
)


def _tc_body(u_ref, p_ref, w_ref, b_ref, o_ref):
    inter = u_ref[...] * p_ref[...]
    o_ref[...] = (
        jnp.dot(inter, w_ref[...], preferred_element_type=jnp.float32)
        + b_ref[...]
    )


_ROWS_BLK = 2048

_tc_logits = pl.pallas_call(
    _tc_body,
    grid=(B // _ROWS_BLK,),
    in_specs=[
        pl.BlockSpec((_ROWS_BLK, D), lambda i: (i, 0)),
        pl.BlockSpec((_ROWS_BLK, D), lambda i: (i, 0)),
        pl.BlockSpec((D, KP), lambda i: (0, 0)),
        pl.BlockSpec((1, KP), lambda i: (0, 0)),
    ],
    out_specs=pl.BlockSpec((_ROWS_BLK, KP), lambda i: (i, 0)),
    out_shape=jax.ShapeDtypeStruct((B, KP), jnp.float32),
)


def kernel(user, product, user_factors, product_factors, W, b):
    user = user.astype(jnp.int32)
    product = product.astype(jnp.int32)
    ug3 = (user >> 3).reshape(NW, NCH, CH)
    us3 = (user & 7).reshape(NW, NCH, CH)
    pg3 = (product >> 3).reshape(NW, NCH, CH)
    ps3 = (product & 7).reshape(NW, NCH, CH)

    uf8 = user_factors.reshape(125000, GL, D)
    pf8 = product_factors.reshape(125000, GL, D)
    u_rows = _sc_gather_rows(ug3, us3, uf8)
    p_rows = _sc_gather_rows(pg3, ps3, pf8)

    wt = jnp.zeros((D, KP), jnp.float32).at[:, :K].set(W.T)
    bp = jnp.zeros((1, KP), jnp.float32).at[0, :K].set(b)
    out = _tc_logits(u_rows, p_rows, wt, bp)
    return out[:, :K]
